# baseline (speedup 1.0000x reference)
import functools

import jax
import jax.numpy as jnp
from jax import lax
from jax.experimental import pallas as pl
from jax.experimental.pallas import tpu as pltpu
from jax.experimental.pallas import tpu_sc as plsc

VOCAB = 1000000
EMBED_DIM = 32
RANK = 8
SCALING = 1.0 / 8.0
PACK = 4
MERGE_BLK = 2000
NBLK = 125
V4 = MERGE_BLK * NBLK


def _merge_body(w0, w1, w2, w3, a0, a1, a2, a3, bsp_ref, p_ref, out_ref):
    parts = []
    for wr, ar in ((w0, a0), (w1, a1), (w2, a2), (w3, a3)):
        parts.append(
            jnp.dot(wr[...], p_ref[...], preferred_element_type=jnp.float32)
            + jnp.dot(ar[...], bsp_ref[...], preferred_element_type=jnp.float32)
        )
    out_ref[...] = jnp.concatenate(parts, axis=1).astype(jnp.bfloat16)


def _merge_table(weight, lora_A, bsp, p):
    w_specs = [
        pl.BlockSpec((MERGE_BLK, EMBED_DIM), lambda i, k=k: (i + k * NBLK, 0))
        for k in range(PACK)
    ]
    a_specs = [
        pl.BlockSpec((MERGE_BLK, RANK), lambda i, k=k: (i + k * NBLK, 0))
        for k in range(PACK)
    ]
    return pl.pallas_call(
        _merge_body,
        grid=(NBLK,),
        in_specs=w_specs
        + a_specs
        + [
            pl.BlockSpec((RANK, EMBED_DIM), lambda i: (0, 0)),
            pl.BlockSpec((EMBED_DIM, EMBED_DIM), lambda i: (0, 0)),
        ],
        out_specs=pl.BlockSpec((MERGE_BLK, PACK * EMBED_DIM), lambda i: (i, 0)),
        out_shape=jax.ShapeDtypeStruct((V4, PACK * EMBED_DIM), jnp.bfloat16),
    )(
        weight, weight, weight, weight,
        lora_A, lora_A, lora_A, lora_A,
        bsp, p,
    )


def _gather_rows(table4, idx_flat, n_groups, group):
    info = plsc.get_sparse_core_info()
    nc, ns = info.num_cores, info.num_subcores
    nw = nc * ns
    gp_w = n_groups // nw
    gp_chunk = 16
    chunk = gp_chunk * group
    n_chunks = gp_w // gp_chunk
    mesh = plsc.VectorSubcoreMesh(core_axis_name="c", subcore_axis_name="s")

    @functools.partial(
        pl.kernel,
        mesh=mesh,
        compiler_params=pltpu.CompilerParams(
            needs_layout_passes=False, use_tc_tiling_on_sc=False
        ),
        out_type=jax.ShapeDtypeStruct((n_groups, group, EMBED_DIM), jnp.float32),
        scratch_types=[
            pltpu.VMEM((chunk,), jnp.int32),
            pltpu.VMEM((chunk,), jnp.int32),
            pltpu.VMEM((chunk,), jnp.int32),
            pltpu.VMEM((chunk,), jnp.int32),
            pltpu.VMEM((chunk,), jnp.int32),
            pltpu.VMEM((chunk,), jnp.int32),
            pltpu.VMEM((chunk, PACK * EMBED_DIM), jnp.bfloat16),
            pltpu.VMEM((chunk, PACK * EMBED_DIM), jnp.bfloat16),
            pltpu.VMEM((gp_chunk, group, EMBED_DIM), jnp.float32),
            pltpu.SemaphoreType.DMA,
            pltpu.SemaphoreType.DMA,
        ],
    )
    def k(
        table_hbm, idx_hbm, out_hbm,
        idx0, idx1, line0, line1, off0, off1, rows0, rows1, out_v,
        sem0, sem1,
    ):
        wid = lax.axis_index("s") * nc + lax.axis_index("c")
        g_base = wid * gp_w
        t_base = g_base * group
        bufs = ((idx0, line0, off0, rows0, sem0), (idx1, line1, off1, rows1, sem1))

        def prep(ci, idx_v, line_v, off_v, rows_v, sem):
            pltpu.sync_copy(idx_hbm.at[pl.ds(t_base + ci * chunk, chunk)], idx_v)

            def split(i, c2):
                v = idx_v[pl.ds(i * 16, 16)]
                one = jnp.full((16,), 1, jnp.int32)
                zero = jnp.full((16,), 0, jnp.int32)
                q = (
                    jnp.where(v >= V4, one, zero)
                    + jnp.where(v >= 2 * V4, one, zero)
                    + jnp.where(v >= 3 * V4, one, zero)
                )
                line = v - q * V4
                line = jnp.minimum(jnp.maximum(line, 0), V4 - 1)
                line_v[pl.ds(i * 16, 16)] = line
                off_v[pl.ds(i * 16, 16)] = lax.shift_left(q, 5)
                return c2

            lax.fori_loop(0, chunk // 16, split, 0, unroll=4)
            pltpu.async_copy(table_hbm.at[line_v], rows_v, sem)

        prep(0, *bufs[0])

        def pair_body(pi, carry):
            for s in range(2):
                ci = pi * 2 + s
                cur = bufs[s]
                nxt = bufs[s ^ 1]

                if s == 0:
                    prep(ci + 1, *nxt)
                else:

                    @pl.when(ci + 1 < n_chunks)
                    def _():
                        prep(ci + 1, *nxt)

                idx_v, line_v, off_v, rows_v, sem = cur
                pltpu.make_async_copy(
                    table_hbm.at[line_v], rows_v, sem
                ).wait()

                def compact16(m, c2):
                    base_t = m * 16
                    qs = off_v[pl.ds(base_t, 16)]
                    for j in range(16):
                        t = base_t + j
                        g = t // group
                        c = lax.rem(t, group)
                        q = pl.multiple_of(qs[j], 32)
                        lo, hi = plsc.unpack(
                            rows_v[t, pl.ds(q, 32)],
                            format=plsc.PackFormat.INTERLEAVED,
                        )
                        out_v[g, c, pl.ds(0, 16)] = lo
                        out_v[g, c, pl.ds(16, 16)] = hi
                    return c2

                lax.fori_loop(0, chunk // 16, compact16, 0)
                pltpu.sync_copy(
                    out_v, out_hbm.at[pl.ds(g_base + ci * gp_chunk, gp_chunk)]
                )
            return carry

        lax.fori_loop(0, n_chunks // 2, pair_body, 0)

    return k(table4, idx_flat)


def kernel(x, weight, lora_A, lora_B):
    d = jnp.arange(EMBED_DIM)
    dst = jnp.where(d < 16, 2 * d, 2 * (d - 16) + 1)
    p = jnp.zeros((EMBED_DIM, EMBED_DIM), jnp.float32).at[d, dst].set(1.0)
    bsp = (lora_B * SCALING) @ p
    merged4 = _merge_table(weight, lora_A, bsp, p)

    b, t, c = x.shape
    flat = x.reshape(-1).astype(jnp.int32)
    out = _gather_rows(merged4, flat, b * t, c)
    return out.reshape(b, t, c, EMBED_DIM)


# speedup vs baseline: 1.0077x; 1.0077x over previous
import functools

import jax
import jax.numpy as jnp
from jax import lax
from jax.experimental import pallas as pl
from jax.experimental.pallas import tpu as pltpu
from jax.experimental.pallas import tpu_sc as plsc

VOCAB = 1000000
EMBED_DIM = 32
RANK = 8
SCALING = 1.0 / 8.0
PACK = 4
MERGE_BLK = 2000
NBLK = 125
V4 = MERGE_BLK * NBLK


def _merge_body(w0, w1, w2, w3, a0, a1, a2, a3, bsp_ref, p_ref, out_ref):
    parts = []
    for wr, ar in ((w0, a0), (w1, a1), (w2, a2), (w3, a3)):
        parts.append(
            jnp.dot(wr[...], p_ref[...], preferred_element_type=jnp.float32)
            + jnp.dot(ar[...], bsp_ref[...], preferred_element_type=jnp.float32)
        )
    out_ref[...] = jnp.concatenate(parts, axis=1).astype(jnp.bfloat16)


def _merge_table(weight, lora_A, bsp, p):
    w_specs = [
        pl.BlockSpec((MERGE_BLK, EMBED_DIM), lambda i, k=k: (i + k * NBLK, 0))
        for k in range(PACK)
    ]
    a_specs = [
        pl.BlockSpec((MERGE_BLK, RANK), lambda i, k=k: (i + k * NBLK, 0))
        for k in range(PACK)
    ]
    return pl.pallas_call(
        _merge_body,
        grid=(NBLK,),
        in_specs=w_specs
        + a_specs
        + [
            pl.BlockSpec((RANK, EMBED_DIM), lambda i: (0, 0)),
            pl.BlockSpec((EMBED_DIM, EMBED_DIM), lambda i: (0, 0)),
        ],
        out_specs=pl.BlockSpec((MERGE_BLK, PACK * EMBED_DIM), lambda i: (i, 0)),
        out_shape=jax.ShapeDtypeStruct((V4, PACK * EMBED_DIM), jnp.bfloat16),
    )(
        weight, weight, weight, weight,
        lora_A, lora_A, lora_A, lora_A,
        bsp, p,
    )


def _gather_rows(table4, x_t):
    nc_dim, nt_dim, nb_dim = x_t.shape
    hb = 512
    n_sub = nb_dim // hb
    slab = EMBED_DIM * nb_dim
    n_units = nc_dim * nt_dim

    info = plsc.get_sparse_core_info()
    nc, ns = info.num_cores, info.num_subcores
    nw = nc * ns
    base_units = n_units // nw
    rem_units = n_units - base_units * nw
    mesh = plsc.VectorSubcoreMesh(core_axis_name="c", subcore_axis_name="s")

    @functools.partial(
        pl.kernel,
        mesh=mesh,
        compiler_params=pltpu.CompilerParams(
            needs_layout_passes=False, use_tc_tiling_on_sc=False
        ),
        out_type=jax.ShapeDtypeStruct((n_units * slab,), jnp.float32),
        scratch_types=[
            pltpu.VMEM((hb,), jnp.int32),
            pltpu.VMEM((hb,), jnp.int32),
            pltpu.VMEM((hb,), jnp.int32),
            pltpu.VMEM((hb,), jnp.int32),
            pltpu.VMEM((hb,), jnp.int32),
            pltpu.VMEM((hb,), jnp.int32),
            pltpu.VMEM((hb, PACK * EMBED_DIM), jnp.bfloat16),
            pltpu.VMEM((hb, PACK * EMBED_DIM), jnp.bfloat16),
            pltpu.VMEM((slab,), jnp.float32),
            pltpu.SemaphoreType.DMA,
            pltpu.SemaphoreType.DMA,
        ],
    )
    def k(
        table_hbm, x_hbm, out_hbm,
        idx0, idx1, line0, line1, off0, off1, rows0, rows1, out_v,
        sem0, sem1,
    ):
        wid = lax.axis_index("s") * nc + lax.axis_index("c")
        n_mine = base_units + jnp.minimum(jnp.maximum(rem_units - wid, 0), 1)
        iota16 = lax.iota(jnp.int32, 16)
        dlo = iota16 * nb_dim
        dhi = dlo + 16 * nb_dim
        bufs = ((idx0, line0, off0, rows0, sem0), (idx1, line1, off1, rows1, sem1))

        def slab_u(ui):
            return ui * nw + wid

        def prep(u, sub, idx_v, line_v, off_v, rows_v, sem):
            tpos = u // nc_dim
            cpos = lax.rem(u, nc_dim)
            pltpu.sync_copy(
                x_hbm.at[cpos, tpos, pl.ds(sub * hb, hb)], idx_v
            )

            def split(i, c2):
                v = idx_v[pl.ds(i * 16, 16)]
                one = jnp.full((16,), 1, jnp.int32)
                zero = jnp.full((16,), 0, jnp.int32)
                q = (
                    jnp.where(v >= V4, one, zero)
                    + jnp.where(v >= 2 * V4, one, zero)
                    + jnp.where(v >= 3 * V4, one, zero)
                )
                line = v - q * V4
                line = jnp.minimum(jnp.maximum(line, 0), V4 - 1)
                line_v[pl.ds(i * 16, 16)] = line
                off_v[pl.ds(i * 16, 16)] = lax.shift_left(q, 5)
                return c2

            lax.fori_loop(0, hb // 16, split, 0, unroll=4)
            pltpu.async_copy(table_hbm.at[line_v], rows_v, sem)

        prep(slab_u(0), 0, *bufs[0])

        def slab_body(ui, carry):
            u = slab_u(ui)
            for sub in range(n_sub):
                cur = bufs[sub]
                nxt = bufs[sub ^ 1]

                if sub == 0:
                    prep(u, 1, *nxt)
                else:

                    @pl.when(ui + 1 < n_mine)
                    def _():
                        prep(slab_u(ui + 1), 0, *nxt)

                idx_v, line_v, off_v, rows_v, sem = cur
                pltpu.make_async_copy(table_hbm.at[line_v], rows_v, sem).wait()
                b0 = sub * hb

                def compact16(m, c2):
                    base_t = m * 16
                    qs = off_v[pl.ds(base_t, 16)]
                    for j in range(16):
                        t = base_t + j
                        q = pl.multiple_of(qs[j], 32)
                        lo, hi = plsc.unpack(
                            rows_v[t, pl.ds(q, 32)],
                            format=plsc.PackFormat.INTERLEAVED,
                        )
                        bl = b0 + t
                        plsc.store_scatter(out_v, [dlo + bl], lo)
                        plsc.store_scatter(out_v, [dhi + bl], hi)
                    return c2

                lax.fori_loop(0, hb // 16, compact16, 0)

            pltpu.sync_copy(out_v, out_hbm.at[pl.ds(u * slab, slab)])
            return carry

        lax.fori_loop(0, n_mine, slab_body, 0)

    return k(table4, x_t)


def kernel(x, weight, lora_A, lora_B):
    d = jnp.arange(EMBED_DIM)
    dst = jnp.where(d < 16, 2 * d, 2 * (d - 16) + 1)
    p = jnp.zeros((EMBED_DIM, EMBED_DIM), jnp.float32).at[d, dst].set(1.0)
    bsp = (lora_B * SCALING) @ p
    merged4 = _merge_table(weight, lora_A, bsp, p)

    b, t, c = x.shape
    x_t = jnp.transpose(x, (2, 1, 0)).astype(jnp.int32)
    out_flat = _gather_rows(merged4, x_t)
    out_phys = out_flat.reshape(t, c, EMBED_DIM, b)
    return jnp.transpose(out_phys, (3, 0, 1, 2))


# speedup vs baseline: 1.2177x; 1.2084x over previous
import functools

import jax
import jax.numpy as jnp
from jax import lax
from jax.experimental import pallas as pl
from jax.experimental.pallas import tpu as pltpu
from jax.experimental.pallas import tpu_sc as plsc

VOCAB = 1000000
EMBED_DIM = 32
RANK = 8
SCALING = 1.0 / 8.0
PACK = 4
MERGE_BLK = 2000
NBLK = 125
V4 = MERGE_BLK * NBLK


def _merge_body(w0, w1, w2, w3, a0, a1, a2, a3, bs_ref, out_ref):
    parts = []
    for wr, ar in ((w0, a0), (w1, a1), (w2, a2), (w3, a3)):
        parts.append(
            wr[...]
            + jnp.dot(ar[...], bs_ref[...], preferred_element_type=jnp.float32)
        )
    out_ref[...] = jnp.concatenate(parts, axis=1)


def _merge_table(weight, lora_A, bs):
    w_specs = [
        pl.BlockSpec((MERGE_BLK, EMBED_DIM), lambda i, k=k: (i + k * NBLK, 0))
        for k in range(PACK)
    ]
    a_specs = [
        pl.BlockSpec((MERGE_BLK, RANK), lambda i, k=k: (i + k * NBLK, 0))
        for k in range(PACK)
    ]
    return pl.pallas_call(
        _merge_body,
        grid=(NBLK,),
        in_specs=w_specs
        + a_specs
        + [pl.BlockSpec((RANK, EMBED_DIM), lambda i: (0, 0))],
        out_specs=pl.BlockSpec((MERGE_BLK, PACK * EMBED_DIM), lambda i: (i, 0)),
        out_shape=jax.ShapeDtypeStruct((V4, PACK * EMBED_DIM), jnp.float32),
    )(
        weight, weight, weight, weight,
        lora_A, lora_A, lora_A, lora_A,
        bs,
    )


def _gather_rows(table4, idx_flat, n_groups, group):
    info = plsc.get_sparse_core_info()
    nc, ns = info.num_cores, info.num_subcores
    nw = nc * ns
    gp_w = n_groups // nw
    gp_chunk = 8
    chunk = gp_chunk * group
    n_chunks = gp_w // gp_chunk
    mesh = plsc.VectorSubcoreMesh(core_axis_name="c", subcore_axis_name="s")

    @functools.partial(
        pl.kernel,
        mesh=mesh,
        out_type=jax.ShapeDtypeStruct((n_groups, group, EMBED_DIM), jnp.float32),
        scratch_types=[
            pltpu.VMEM((chunk,), jnp.int32),
            pltpu.VMEM((chunk,), jnp.int32),
            pltpu.VMEM((chunk,), jnp.int32),
            pltpu.VMEM((chunk,), jnp.int32),
            pltpu.VMEM((chunk,), jnp.int32),
            pltpu.VMEM((chunk,), jnp.int32),
            pltpu.VMEM((chunk, PACK * EMBED_DIM), jnp.float32),
            pltpu.VMEM((chunk, PACK * EMBED_DIM), jnp.float32),
            pltpu.VMEM((gp_chunk, group, EMBED_DIM), jnp.float32),
            pltpu.SemaphoreType.DMA,
            pltpu.SemaphoreType.DMA,
        ],
    )
    def k(
        table_hbm, idx_hbm, out_hbm,
        idx0, idx1, line0, line1, off0, off1, rows0, rows1, out_v,
        sem0, sem1,
    ):
        wid = lax.axis_index("s") * nc + lax.axis_index("c")
        g_base = wid * gp_w
        t_base = g_base * group
        bufs = ((idx0, line0, off0, rows0, sem0), (idx1, line1, off1, rows1, sem1))

        def prep(ci, idx_v, line_v, off_v, rows_v, sem):
            pltpu.sync_copy(idx_hbm.at[pl.ds(t_base + ci * chunk, chunk)], idx_v)

            def split(i, c2):
                v = idx_v[pl.ds(i * 16, 16)]
                one = jnp.full((16,), 1, jnp.int32)
                zero = jnp.full((16,), 0, jnp.int32)
                q = (
                    jnp.where(v >= V4, one, zero)
                    + jnp.where(v >= 2 * V4, one, zero)
                    + jnp.where(v >= 3 * V4, one, zero)
                )
                line_v[pl.ds(i * 16, 16)] = v - q * V4
                off_v[pl.ds(i * 16, 16)] = lax.shift_left(q, 5)
                return c2

            lax.fori_loop(0, chunk // 16, split, 0, unroll=4)
            pltpu.async_copy(table_hbm.at[line_v], rows_v, sem)

        prep(0, *bufs[0])

        def pair_body(pi, carry):
            for s in range(2):
                ci = pi * 2 + s
                cur = bufs[s]
                nxt = bufs[s ^ 1]

                if s == 0:
                    prep(ci + 1, *nxt)
                else:

                    @pl.when(ci + 1 < n_chunks)
                    def _():
                        prep(ci + 1, *nxt)

                idx_v, line_v, off_v, rows_v, sem = cur
                pltpu.make_async_copy(table_hbm.at[line_v], rows_v, sem).wait()

                def compact16(m, c2):
                    base_t = m * 16
                    qs = off_v[pl.ds(base_t, 16)]
                    for j in range(16):
                        t = base_t + j
                        g = t // group
                        c = lax.rem(t, group)
                        q = qs[j]
                        out_v[g, c, pl.ds(0, 16)] = rows_v[t, pl.ds(q, 16)]
                        out_v[g, c, pl.ds(16, 16)] = rows_v[t, pl.ds(q + 16, 16)]
                    return c2

                lax.fori_loop(0, chunk // 16, compact16, 0)
                pltpu.sync_copy(
                    out_v, out_hbm.at[pl.ds(g_base + ci * gp_chunk, gp_chunk)]
                )
            return carry

        lax.fori_loop(0, n_chunks // 2, pair_body, 0)

    return k(table4, idx_flat)


def kernel(x, weight, lora_A, lora_B):
    merged4 = _merge_table(weight, lora_A, lora_B * SCALING)
    b, t, c = x.shape
    flat = x.reshape(-1).astype(jnp.int32)
    out = _gather_rows(merged4, flat, b * t, c)
    return out.reshape(b, t, c, EMBED_DIM)
